# baseline (speedup 1.0000x reference)
import functools

import jax
import jax.numpy as jnp
import numpy as np
from jax import lax
from jax.experimental import pallas as pl
from jax.experimental.pallas import tpu as pltpu
from jax.experimental.pallas import tpu_sc as plsc

_NBITS = 2

_N = 2 * 4096 * 2048
_NC, _NS, _L = 2, 16, 16
_NW = _NC * _NS
_PER_W = _N // _NW
_C = 16384
_NCHUNK = _PER_W // _C


def _enc_matrix():
    bitvecs = np.unpackbits(
        np.arange(2 ** _NBITS, dtype=np.uint8).reshape(-1, 1), axis=1
    )[:, -_NBITS:]
    return jnp.asarray(bitvecs.astype(np.float32) * 2.0 - 1.0)


def _sc_body(p_hbm, x_hbm, o_hbm, p_v, in_v, out_v,
             isem0, isem1, osem0, osem1):
    wid = lax.axis_index("s") * _NC + lax.axis_index("c")
    base = wid * _PER_W

    pltpu.sync_copy(p_hbm, p_v)
    l0 = p_v[0]
    l1 = p_v[1]
    l2 = p_v[2]
    l3 = p_v[3]
    t0 = p_v[4]
    t1 = p_v[5]
    t2 = p_v[6]

    isems = (isem0, isem1)
    osems = (osem0, osem1)

    def in_copy(c, slot, sem):
        return pltpu.async_copy(
            x_hbm.at[pl.ds(base + c * _C, _C)], in_v.at[slot], sem)

    def out_copy(c, slot, sem):
        return pltpu.async_copy(
            out_v.at[slot], o_hbm.at[pl.ds(base + c * _C, _C)], sem)

    in_copy(0, 0, isems[0])

    def pair_body(pair, carry):
        for b in range(2):
            c = pair * 2 + b
            pltpu.make_async_copy(
                x_hbm.at[pl.ds(0, _C)], in_v.at[b], isems[b]).wait()
            @pl.when(c + 1 < _NCHUNK)
            def _():
                in_copy(c + 1, 1 - b, isems[1 - b])
            @pl.when(c >= 2)
            def _():
                pltpu.make_async_copy(
                    out_v.at[b], o_hbm.at[pl.ds(0, _C)], osems[b]).wait()

            def vec_body(j, _):
                v = in_v[b, pl.ds(j * _L, _L)]
                lo = jnp.where(v > t0, l1, l0)
                hi = jnp.where(v > t2, l3, l2)
                out_v[b, pl.ds(j * _L, _L)] = jnp.where(v > t1, hi, lo)
                return 0

            lax.fori_loop(0, _C // _L, vec_body, 0, unroll=8)
            out_copy(c, b, osems[b])
        return carry

    lax.fori_loop(0, _NCHUNK // 2, pair_body, 0)

    for b in range(2):
        pltpu.make_async_copy(
            out_v.at[b], o_hbm.at[pl.ds(0, _C)], osems[b]).wait()


def kernel(x, basis):
    qlevels = jnp.sort(_enc_matrix() @ basis)
    thres = (qlevels[:-1] + qlevels[1:]) * 0.5
    params = jnp.broadcast_to(
        jnp.concatenate([qlevels, thres])[:, None], (7, _L))

    xf = x.reshape(_N)
    mesh = plsc.VectorSubcoreMesh(core_axis_name="c", subcore_axis_name="s")

    run = pl.kernel(
        _sc_body,
        mesh=mesh,
        out_type=jax.ShapeDtypeStruct((_N,), jnp.float32),
        scratch_types=[
            pltpu.VMEM((7, _L), jnp.float32),
            pltpu.VMEM((2, _C), jnp.float32),
            pltpu.VMEM((2, _C), jnp.float32),
            pltpu.SemaphoreType.DMA,
            pltpu.SemaphoreType.DMA,
            pltpu.SemaphoreType.DMA,
            pltpu.SemaphoreType.DMA,
        ],
    )
    out = run(params, xf)
    return out.reshape(x.shape)


# speedup vs baseline: 1.5963x; 1.5963x over previous
import functools

import jax
import jax.numpy as jnp
import numpy as np
from jax import lax
from jax.experimental import pallas as pl
from jax.experimental.pallas import tpu as pltpu
from jax.experimental.pallas import tpu_sc as plsc

_NBITS = 2

_N = 2 * 4096 * 2048
_NC, _NS, _L = 2, 16, 16
_NW = _NC * _NS
_PER_W = _N // _NW
_C = 16384
_NCHUNK = _PER_W // _C


def _enc_matrix():
    bitvecs = np.unpackbits(
        np.arange(2 ** _NBITS, dtype=np.uint8).reshape(-1, 1), axis=1
    )[:, -_NBITS:]
    return jnp.asarray(bitvecs.astype(np.float32) * 2.0 - 1.0)


def _sc_body(p_hbm, x_hbm, o_hbm, p_v, in_v, out_v,
             isem0, isem1, osem0, osem1):
    wid = lax.axis_index("s") * _NC + lax.axis_index("c")
    base = wid * _PER_W

    pltpu.sync_copy(p_hbm, p_v)
    l0 = p_v[0]
    l1 = p_v[1]
    l2 = p_v[2]
    l3 = p_v[3]
    t0 = p_v[4]
    t1 = p_v[5]
    t2 = p_v[6]

    isems = (isem0, isem1)
    osems = (osem0, osem1)

    def in_copy(c, slot, sem):
        return pltpu.async_copy(
            x_hbm.at[pl.ds(base + c * _C, _C)], in_v.at[slot], sem)

    def out_copy(c, slot, sem):
        return pltpu.async_copy(
            out_v.at[slot], o_hbm.at[pl.ds(base + c * _C, _C)], sem)

    in_copy(0, 0, isems[0])

    def pair_body(pair, carry):
        for b in range(2):
            c = pair * 2 + b
            pltpu.make_async_copy(
                x_hbm.at[pl.ds(0, _C)], in_v.at[b], isems[b]).wait()
            @pl.when(c + 1 < _NCHUNK)
            def _():
                in_copy(c + 1, 1 - b, isems[1 - b])
            @pl.when(c >= 2)
            def _():
                pltpu.make_async_copy(
                    out_v.at[b], o_hbm.at[pl.ds(0, _C)], osems[b]).wait()

            @plsc.parallel_loop(0, _C // _L, unroll=8)
            def _(j):
                v = in_v[b, pl.ds(j * _L, _L)]
                lo = jnp.where(v > t0, l1, l0)
                hi = jnp.where(v > t2, l3, l2)
                out_v[b, pl.ds(j * _L, _L)] = jnp.where(v > t1, hi, lo)
            out_copy(c, b, osems[b])
        return carry

    lax.fori_loop(0, _NCHUNK // 2, pair_body, 0)

    for b in range(2):
        pltpu.make_async_copy(
            out_v.at[b], o_hbm.at[pl.ds(0, _C)], osems[b]).wait()


def kernel(x, basis):
    qlevels = jnp.sort(_enc_matrix() @ basis)
    thres = (qlevels[:-1] + qlevels[1:]) * 0.5
    params = jnp.broadcast_to(
        jnp.concatenate([qlevels, thres])[:, None], (7, _L))

    xf = x.reshape(_N)
    mesh = plsc.VectorSubcoreMesh(core_axis_name="c", subcore_axis_name="s")

    run = pl.kernel(
        _sc_body,
        mesh=mesh,
        out_type=jax.ShapeDtypeStruct((_N,), jnp.float32),
        scratch_types=[
            pltpu.VMEM((7, _L), jnp.float32),
            pltpu.VMEM((2, _C), jnp.float32),
            pltpu.VMEM((2, _C), jnp.float32),
            pltpu.SemaphoreType.DMA,
            pltpu.SemaphoreType.DMA,
            pltpu.SemaphoreType.DMA,
            pltpu.SemaphoreType.DMA,
        ],
    )
    out = run(params, xf)
    return out.reshape(x.shape)


# speedup vs baseline: 4.5010x; 2.8197x over previous
import functools

import jax
import jax.numpy as jnp
import numpy as np
from jax import lax
from jax.experimental import pallas as pl
from jax.experimental.pallas import tpu as pltpu
from jax.experimental.pallas import tpu_sc as plsc

_NBITS = 2

_ROWS, _COLS = 8192, 2048
_NC, _NS, _L = 2, 16, 16
_NW = _NC * _NS
_ROWS_W = _ROWS // _NW
_CR = 8
_NCHUNK = _ROWS_W // _CR


def _enc_matrix():
    bitvecs = np.unpackbits(
        np.arange(2 ** _NBITS, dtype=np.uint8).reshape(-1, 1), axis=1
    )[:, -_NBITS:]
    return jnp.asarray(bitvecs.astype(np.float32) * 2.0 - 1.0)


def _sc_body(p_hbm, x_hbm, o_hbm, p_v, in_v, out_v,
             isem0, isem1, osem0, osem1):
    wid = lax.axis_index("s") * _NC + lax.axis_index("c")
    base = wid * _ROWS_W

    pltpu.sync_copy(p_hbm, p_v)
    l0 = p_v[0]
    l1 = p_v[1]
    l2 = p_v[2]
    l3 = p_v[3]
    t0 = p_v[4]
    t1 = p_v[5]
    t2 = p_v[6]

    isems = (isem0, isem1)
    osems = (osem0, osem1)

    def in_copy(c, slot, sem):
        return pltpu.async_copy(
            x_hbm.at[pl.ds(base + c * _CR, _CR), :], in_v.at[slot], sem)

    def out_copy(c, slot, sem):
        return pltpu.async_copy(
            out_v.at[slot], o_hbm.at[pl.ds(base + c * _CR, _CR), :], sem)

    in_copy(0, 0, isems[0])

    def pair_body(pair, carry):
        for b in range(2):
            c = pair * 2 + b
            pltpu.make_async_copy(
                x_hbm.at[pl.ds(0, _CR), :], in_v.at[b], isems[b]).wait()
            @pl.when(c + 1 < _NCHUNK)
            def _():
                in_copy(c + 1, 1 - b, isems[1 - b])
            @pl.when(c >= 2)
            def _():
                pltpu.make_async_copy(
                    out_v.at[b], o_hbm.at[pl.ds(0, _CR), :], osems[b]).wait()

            for r in range(_CR):
                @plsc.parallel_loop(0, _COLS // _L, unroll=8)
                def _(j):
                    v = in_v[b, r, pl.ds(j * _L, _L)]
                    lo = jnp.where(v > t0, l1, l0)
                    hi = jnp.where(v > t2, l3, l2)
                    out_v[b, r, pl.ds(j * _L, _L)] = jnp.where(v > t1, hi, lo)

            out_copy(c, b, osems[b])
        return carry

    lax.fori_loop(0, _NCHUNK // 2, pair_body, 0)

    for b in range(2):
        pltpu.make_async_copy(
            out_v.at[b], o_hbm.at[pl.ds(0, _CR), :], osems[b]).wait()


def kernel(x, basis):
    qlevels = jnp.sort(_enc_matrix() @ basis)
    thres = (qlevels[:-1] + qlevels[1:]) * 0.5
    params = jnp.broadcast_to(
        jnp.concatenate([qlevels, thres])[:, None], (7, _L))

    xf = x.reshape(_ROWS, _COLS)
    mesh = plsc.VectorSubcoreMesh(core_axis_name="c", subcore_axis_name="s")

    run = pl.kernel(
        _sc_body,
        mesh=mesh,
        out_type=jax.ShapeDtypeStruct((_ROWS, _COLS), jnp.float32),
        compiler_params=pltpu.CompilerParams(use_tc_tiling_on_sc=True),
        scratch_types=[
            pltpu.VMEM((7, _L), jnp.float32),
            pltpu.VMEM((2, _CR, _COLS), jnp.float32),
            pltpu.VMEM((2, _CR, _COLS), jnp.float32),
            pltpu.SemaphoreType.DMA,
            pltpu.SemaphoreType.DMA,
            pltpu.SemaphoreType.DMA,
            pltpu.SemaphoreType.DMA,
        ],
    )
    out = run(params, xf)
    return out.reshape(x.shape)
